# baseline (speedup 1.0000x reference)
import functools

import numpy as np
import jax
import jax.numpy as jnp
from jax import lax
from jax.experimental import pallas as pl
from jax.experimental.pallas import tpu as pltpu
from jax.experimental.pallas import tpu_sc as plsc

_SCALING = float(np.power(2.0, np.random.default_rng(seed=42).uniform(-1, 1)))
_L = 220500
_OUTPUT_SIZE = int(_L * _SCALING)
_OFFSET = (_OUTPUT_SIZE - _L) // 2
_SCALING_F32 = np.float32(_SCALING)

_NC, _NS = 2, 16
_NW = _NC * _NS
_ROWS = 16
_C = 6912
_PAD = _C * _NW
_ITERS = _C // 16
_W0 = 34560
_WSTRIDE = 4736
_WIN = 5120


def _interp_body(wav_hbm, out_hbm, win_v, out_v, src_v, sin, sout):
    wid = lax.axis_index("s") * _NC + lax.axis_index("c")
    start = _W0 + wid * _WSTRIDE
    jbase0 = wid * _C
    lane = lax.iota(jnp.int32, 16)

    def win_copy(b, buf):
        return pltpu.async_copy(
            wav_hbm.at[1, b, :, pl.ds(start, _WIN)], win_v.at[buf], sin.at[buf]
        )

    in_dma = [win_copy(0, 0), win_copy(1, 1)]

    @plsc.parallel_loop(0, _C, 16, unroll=4)
    def _pre(i):
        jv = jbase0 + i + lane
        src_v[pl.ds(i, 16)] = (jv + _OFFSET).astype(jnp.float32) / _SCALING_F32

    out_dma = [None, None]
    for b in range(8):
        in_dma[b % 2].wait()
        if out_dma[b % 2] is not None:
            out_dma[b % 2].wait()
        for ch in range(2):
            chs = jnp.full((16,), ch, jnp.int32)

            @plsc.parallel_loop(0, _C, 16, unroll=8)
            def _body(i, ch=ch, chs=chs, b2=b % 2):
                src = src_v[pl.ds(i, 16)]
                i0 = src.astype(jnp.int32)
                r = src - i0.astype(jnp.float32)
                li0 = i0 - start
                x0 = plsc.load_gather(win_v.at[b2], [chs, li0])
                x1 = plsc.load_gather(win_v.at[b2], [chs, li0 + 1])
                out_v[b % 2, ch, pl.ds(i, 16)] = x0 + r * (x1 - x0)

        if b + 2 < 8:
            in_dma[b % 2] = win_copy(b + 2, b % 2)
        out_dma[b % 2] = pltpu.async_copy(
            out_v.at[b % 2], out_hbm.at[b, :, pl.ds(jbase0, _C)], sout.at[b % 2]
        )
    out_dma[0].wait()
    out_dma[1].wait()


@jax.jit
def _sc_interp(wav):
    mesh = plsc.VectorSubcoreMesh(core_axis_name="c", subcore_axis_name="s")
    f = functools.partial(
        pl.kernel,
        mesh=mesh,
        out_type=jax.ShapeDtypeStruct((8, 2, _PAD), jnp.float32),
        scratch_types=[
            pltpu.VMEM((2, 2, _WIN), jnp.float32),
            pltpu.VMEM((2, 2, _C), jnp.float32),
            pltpu.VMEM((_C,), jnp.float32),
            pltpu.SemaphoreType.DMA((2,)),
            pltpu.SemaphoreType.DMA((2,)),
        ],
        compiler_params=pltpu.CompilerParams(needs_layout_passes=False),
    )(_interp_body)
    return f(wav)


def _copy023_body(wav_ref, out_ref):
    out_ref[...] = wav_ref[...]


@jax.jit
def _copy023(wav):
    return pl.pallas_call(
        _copy023_body,
        out_shape=jax.ShapeDtypeStruct((4, 8, 2, _L), jnp.float32),
        grid=(3, 4),
        in_specs=[
            pl.BlockSpec((1, 2, 2, _L), lambda t, b: (t + (t >= 1), b, 0, 0)),
        ],
        out_specs=pl.BlockSpec((1, 2, 2, _L), lambda t, b: (t + (t >= 1), b, 0, 0)),
    )(wav)


def _insert_body(base_ref, scaled_ref, out_ref):
    out_ref[0] = scaled_ref[:, :, :_L]


@jax.jit
def _insert(base, scaled):
    return pl.pallas_call(
        _insert_body,
        out_shape=jax.ShapeDtypeStruct((4, 8, 2, _L), jnp.float32),
        grid=(8,),
        in_specs=[
            pl.BlockSpec(memory_space=pl.ANY),
            pl.BlockSpec((1, 2, _PAD), lambda b: (b, 0, 0)),
        ],
        out_specs=pl.BlockSpec((1, 1, 2, _L), lambda b: (1, b, 0, 0)),
        input_output_aliases={0: 0},
    )(base, scaled)


def kernel(wav):
    scaled = _sc_interp(wav)
    return _insert(_copy023(wav), scaled)


# speedup vs baseline: 1.0700x; 1.0700x over previous
import functools

import numpy as np
import jax
import jax.numpy as jnp
from jax import lax
from jax.experimental import pallas as pl
from jax.experimental.pallas import tpu as pltpu
from jax.experimental.pallas import tpu_sc as plsc

_SCALING = float(np.power(2.0, np.random.default_rng(seed=42).uniform(-1, 1)))
_L = 220500
_OUTPUT_SIZE = int(_L * _SCALING)
_OFFSET = (_OUTPUT_SIZE - _L) // 2
_SCALING_F32 = np.float32(_SCALING)

_NC, _NS = 2, 16
_NW = _NC * _NS
_ROWS = 16
_C = 6912
_PAD = _C * _NW
_ITERS = _C // 16
_W0 = 34560
_WSTRIDE = 4736
_WIN = 5120


def _interp_body(wav_hbm, out_hbm, win_v, out_v, src_v, sin, sout):
    wid = lax.axis_index("s") * _NC + lax.axis_index("c")
    start = _W0 + wid * _WSTRIDE
    jbase0 = wid * _C
    lane = lax.iota(jnp.int32, 16)

    def win_copy(b, buf):
        return pltpu.async_copy(
            wav_hbm.at[1, b, :, pl.ds(start, _WIN)], win_v.at[buf], sin.at[buf]
        )

    in_dma = [win_copy(0, 0), win_copy(1, 1)]

    @plsc.parallel_loop(0, _C, 16, unroll=4)
    def _pre(i):
        jv = jbase0 + i + lane
        src_v[pl.ds(i, 16)] = (jv + _OFFSET).astype(jnp.float32) / _SCALING_F32

    out_dma = [None, None]
    for b in range(8):
        in_dma[b % 2].wait()
        if out_dma[b % 2] is not None:
            out_dma[b % 2].wait()
        for ch in range(2):
            chs = jnp.full((16,), ch, jnp.int32)

            @plsc.parallel_loop(0, _C, 16, unroll=4)
            def _body(i, ch=ch, chs=chs, b2=b % 2):
                src = src_v[pl.ds(i, 16)]
                i0 = src.astype(jnp.int32)
                r = src - i0.astype(jnp.float32)
                li0 = i0 - start
                x0 = plsc.load_gather(win_v.at[b2], [chs, li0])
                x1 = plsc.load_gather(win_v.at[b2], [chs, li0 + 1])
                out_v[b % 2, ch, pl.ds(i, 16)] = x0 + r * (x1 - x0)

        if b + 2 < 8:
            in_dma[b % 2] = win_copy(b + 2, b % 2)
        out_dma[b % 2] = pltpu.async_copy(
            out_v.at[b % 2], out_hbm.at[b, :, pl.ds(jbase0, _C)], sout.at[b % 2]
        )
    out_dma[0].wait()
    out_dma[1].wait()


@jax.jit
def _sc_interp(wav):
    mesh = plsc.VectorSubcoreMesh(core_axis_name="c", subcore_axis_name="s")
    f = functools.partial(
        pl.kernel,
        mesh=mesh,
        out_type=jax.ShapeDtypeStruct((8, 2, _PAD), jnp.float32),
        scratch_types=[
            pltpu.VMEM((2, 2, _WIN), jnp.float32),
            pltpu.VMEM((2, 2, _C), jnp.float32),
            pltpu.VMEM((_C,), jnp.float32),
            pltpu.SemaphoreType.DMA((2,)),
            pltpu.SemaphoreType.DMA((2,)),
        ],
        compiler_params=pltpu.CompilerParams(needs_layout_passes=False),
    )(_interp_body)
    return f(wav)


def _copy023_body(wav_ref, out_ref):
    out_ref[...] = wav_ref[...]


@jax.jit
def _copy023(wav):
    return pl.pallas_call(
        _copy023_body,
        out_shape=jax.ShapeDtypeStruct((4, 8, 2, _L), jnp.float32),
        grid=(3, 8),
        in_specs=[
            pl.BlockSpec((1, 1, 2, _L), lambda t, b: (t + (t >= 1), b, 0, 0)),
        ],
        out_specs=pl.BlockSpec((1, 1, 2, _L), lambda t, b: (t + (t >= 1), b, 0, 0)),
    )(wav)


def _insert_body(base_ref, scaled_ref, out_ref):
    out_ref[0] = scaled_ref[:, :, :_L]


@jax.jit
def _insert(base, scaled):
    return pl.pallas_call(
        _insert_body,
        out_shape=jax.ShapeDtypeStruct((4, 8, 2, _L), jnp.float32),
        grid=(8,),
        in_specs=[
            pl.BlockSpec(memory_space=pl.ANY),
            pl.BlockSpec((1, 2, _PAD), lambda b: (b, 0, 0)),
        ],
        out_specs=pl.BlockSpec((1, 1, 2, _L), lambda b: (1, b, 0, 0)),
        input_output_aliases={0: 0},
    )(base, scaled)


def kernel(wav):
    scaled = _sc_interp(wav)
    return _insert(_copy023(wav), scaled)
